# baseline (speedup 1.0000x reference)
import functools

import jax
import jax.numpy as jnp
from jax.experimental import pallas as pl
from jax.experimental.pallas import tpu as pltpu

N_NODES = 50000
OUT = 512
NUM_GRAPHS = 256
BLK = 1000


def _gin_mlp_body(v_ref, a_ref, w1_ref, b1_ref, w2_ref, b2_ref,
                  u_ref, s_ref, q_ref):
    t = v_ref[...] + a_ref[...]
    h = jnp.maximum(
        jax.lax.dot(t, w1_ref[...], preferred_element_type=jnp.float32)
        + b1_ref[...], 0.0)
    u = jnp.maximum(
        jax.lax.dot(h, w2_ref[...], preferred_element_type=jnp.float32)
        + b2_ref[...], 0.0)
    u_ref[...] = u

    @pl.when(pl.program_id(0) == 0)
    def _():
        s_ref[...] = jnp.zeros_like(s_ref)
        q_ref[...] = jnp.zeros_like(q_ref)

    s_ref[...] += jnp.sum(u, axis=0, keepdims=True)
    q_ref[...] += jnp.sum(u * u, axis=0, keepdims=True)


def _gin_mlp(v, agg, w1, b1, w2, b2):
    n, din = v.shape
    grid = (n // BLK,)
    u, s, q = pl.pallas_call(
        _gin_mlp_body,
        grid=grid,
        in_specs=[
            pl.BlockSpec((BLK, din), lambda i: (i, 0)),
            pl.BlockSpec((BLK, din), lambda i: (i, 0)),
            pl.BlockSpec((din, OUT), lambda i: (0, 0)),
            pl.BlockSpec((1, OUT), lambda i: (0, 0)),
            pl.BlockSpec((OUT, OUT), lambda i: (0, 0)),
            pl.BlockSpec((1, OUT), lambda i: (0, 0)),
        ],
        out_specs=[
            pl.BlockSpec((BLK, OUT), lambda i: (i, 0)),
            pl.BlockSpec((1, OUT), lambda i: (0, 0)),
            pl.BlockSpec((1, OUT), lambda i: (0, 0)),
        ],
        out_shape=[
            jax.ShapeDtypeStruct((n, OUT), jnp.float32),
            jax.ShapeDtypeStruct((1, OUT), jnp.float32),
            jax.ShapeDtypeStruct((1, OUT), jnp.float32),
        ],
    )(v, agg, w1, b1.reshape(1, OUT), w2, b2.reshape(1, OUT))
    return u, s, q


def _bn_coeffs_body(s_ref, q_ref, g_ref, b_ref, a_ref, c_ref, *, n_rows):
    m = s_ref[...] / n_rows
    var = q_ref[...] / n_rows - m * m
    a = g_ref[...] * jax.lax.rsqrt(var + 1e-5)
    a_ref[...] = a
    c_ref[...] = b_ref[...] - m * a


def _bn_coeffs(s, q, g, b, n_rows):
    a, c = pl.pallas_call(
        functools.partial(_bn_coeffs_body, n_rows=float(n_rows)),
        out_shape=[
            jax.ShapeDtypeStruct((1, OUT), jnp.float32),
            jax.ShapeDtypeStruct((1, OUT), jnp.float32),
        ],
    )(s, q, g.reshape(1, OUT), b.reshape(1, OUT))
    return a, c


def _bn_apply_body(u_ref, a_ref, c_ref, v_ref):
    v_ref[...] = u_ref[...] * a_ref[...] + c_ref[...]


def _bn_apply(u, a, c, blk):
    n = u.shape[0]
    return pl.pallas_call(
        _bn_apply_body,
        grid=(n // blk,),
        in_specs=[
            pl.BlockSpec((blk, OUT), lambda i: (i, 0)),
            pl.BlockSpec((1, OUT), lambda i: (0, 0)),
            pl.BlockSpec((1, OUT), lambda i: (0, 0)),
        ],
        out_specs=pl.BlockSpec((blk, OUT), lambda i: (i, 0)),
        out_shape=jax.ShapeDtypeStruct((n, OUT), jnp.float32),
    )(u, a, c)


def _branch_body(f_ref, w_ref, b_ref, u_ref, s_ref, q_ref):
    u = jnp.maximum(
        jax.lax.dot(f_ref[...], w_ref[...], preferred_element_type=jnp.float32)
        + b_ref[...], 0.0)
    u_ref[...] = u

    @pl.when(pl.program_id(0) == 0)
    def _():
        s_ref[...] = jnp.zeros_like(s_ref)
        q_ref[...] = jnp.zeros_like(q_ref)

    s_ref[...] += jnp.sum(u, axis=0, keepdims=True)
    q_ref[...] += jnp.sum(u * u, axis=0, keepdims=True)


def _branch(f, w, b, blk=512):
    n, din = f.shape
    u, s, q = pl.pallas_call(
        _branch_body,
        grid=(n // blk,),
        in_specs=[
            pl.BlockSpec((blk, din), lambda i: (i, 0)),
            pl.BlockSpec((din, OUT), lambda i: (0, 0)),
            pl.BlockSpec((1, OUT), lambda i: (0, 0)),
        ],
        out_specs=[
            pl.BlockSpec((blk, OUT), lambda i: (i, 0)),
            pl.BlockSpec((1, OUT), lambda i: (0, 0)),
            pl.BlockSpec((1, OUT), lambda i: (0, 0)),
        ],
        out_shape=[
            jax.ShapeDtypeStruct((n, OUT), jnp.float32),
            jax.ShapeDtypeStruct((1, OUT), jnp.float32),
            jax.ShapeDtypeStruct((1, OUT), jnp.float32),
        ],
    )(f, w, b.reshape(1, OUT))
    return u, s, q


def _fc_body(p_ref, w_ref, b_ref, o_ref):
    o_ref[...] = jnp.maximum(
        jax.lax.dot(p_ref[...], w_ref[...], preferred_element_type=jnp.float32)
        + b_ref[...], 0.0)


def _fc(p, w, b):
    return pl.pallas_call(
        _fc_body,
        out_shape=jax.ShapeDtypeStruct((NUM_GRAPHS, OUT), jnp.float32),
    )(p, w, b.reshape(1, OUT))


def kernel(x, edge_index, batch, mic_feature, dis_feature, c1W1, c1b1, c1W2, c1b2, c2W1, c2b1, c2W2, c2b2, c3W1, c3b1, c3W2, c3b2, bn1_g, bn1_b, bn2_g, bn2_b, bn3_g, bn3_b, fc1_W, fc1_b, dis_W, dis_b, bn_dis_g, bn_dis_b, mic_W, mic_b, bn_mic_g, bn_mic_b):
    src = edge_index[0]
    dst = edge_index[1]

    def agg(v):
        return jnp.zeros(v.shape, v.dtype).at[dst].add(v[src])

    v = x
    for (w1, b1, w2, b2, g, b) in (
            (c1W1, c1b1, c1W2, c1b2, bn1_g, bn1_b),
            (c2W1, c2b1, c2W2, c2b2, bn2_g, bn2_b),
            (c3W1, c3b1, c3W2, c3b2, bn3_g, bn3_b)):
        u, s, q = _gin_mlp(v, agg(v), w1, b1, w2, b2)
        a, c = _bn_coeffs(s, q, g, b, N_NODES)
        v = _bn_apply(u, a, c, BLK)

    pooled = jax.ops.segment_max(v, batch, num_segments=NUM_GRAPHS)
    x_d = _fc(pooled, fc1_W, fc1_b)

    u_dis, s_dis, q_dis = _branch(dis_feature, dis_W, dis_b)
    a_dis, c_dis = _bn_coeffs(s_dis, q_dis, bn_dis_g, bn_dis_b, u_dis.shape[0])
    x_dis = _bn_apply(u_dis, a_dis, c_dis, 512)

    u_mic, s_mic, q_mic = _branch(mic_feature, mic_W, mic_b)
    a_mic, c_mic = _bn_coeffs(s_mic, q_mic, bn_mic_g, bn_mic_b, u_mic.shape[0])
    x_mic = _bn_apply(u_mic, a_mic, c_mic, 512)

    return (x_d, x_mic, x_dis)
